# baseline (speedup 1.0000x reference)
import functools

import jax
import jax.numpy as jnp
from jax.experimental import pallas as pl
from jax.experimental.pallas import tpu as pltpu


def _attend(qkv_ref, att_ref, group, i0, *, n_group_heads, hs, rows, R, T):
    row = jax.lax.broadcasted_iota(jnp.int32, (R, R), 0)
    col = jax.lax.broadcasted_iota(jnp.int32, (R, R), 1)
    keep = (col <= row) & ((row // T) == (col // T))
    neg_big = jnp.float32(-1e30)

    for b in range(rows // R):
        r0 = i0 + b * R
        for hh in range(n_group_heads):
            lane = hh * hs
            q = qkv_ref[3 * group + 0, pl.ds(r0, R), lane:lane + hs]
            k = qkv_ref[3 * group + 1, pl.ds(r0, R), lane:lane + hs]
            v = qkv_ref[3 * group + 2, pl.ds(r0, R), lane:lane + hs]

            s = jax.lax.dot_general(q, k,
                                    dimension_numbers=(((1,), (1,)), ((), ())),
                                    preferred_element_type=jnp.float32)
            s = jnp.where(keep, s, neg_big)
            s = s - jnp.max(s, axis=-1, keepdims=True)
            p = jnp.exp(s)
            p = p * pl.reciprocal(jnp.sum(p, axis=-1, keepdims=True),
                                  approx=True)
            o_hb = jnp.dot(p.astype(jnp.bfloat16), v,
                           preferred_element_type=jnp.float32)
            gl = group * n_group_heads * hs + lane
            att_ref[pl.ds(r0, R), gl:gl + hs] = o_hb.astype(jnp.bfloat16)


def _mha_body(x_ref, wqkv_ref, wpt_ref, bp_ref, o_ref,
              x_bf_ref, qkv_ref, wpt_sc_ref, att_ref,
              *, num_heads, seq_len, seqs_per_block, n_strips):
    j = pl.program_id(0)
    BT, C = x_ref.shape
    hs = C // num_heads
    T = seq_len
    R = seqs_per_block * T
    half = BT // 2
    ngh = num_heads // 2

    @pl.when(j < n_strips)
    def _():
        @pl.when(j == 0)
        def _():
            x_bf_ref[...] = x_ref[...].astype(jnp.bfloat16)

        strip = jnp.dot(x_bf_ref[...], wqkv_ref[...],
                        preferred_element_type=jnp.float32)
        qkv_ref[j] = strip.astype(jnp.bfloat16)

        wpt_sc_ref[pl.ds(j * (C // n_strips), C // n_strips), :] = wpt_ref[...]

    @pl.when((j >= 3) & (j < 5))
    def _():
        _attend(qkv_ref, att_ref, 0, (j - 3) * half,
                n_group_heads=ngh, hs=hs, rows=half, R=R, T=T)

    @pl.when((j >= 6) & (j < 8))
    def _():
        _attend(qkv_ref, att_ref, 1, (j - 6) * half,
                n_group_heads=ngh, hs=hs, rows=half, R=R, T=T)

    @pl.when(j >= 7)
    def _():
        cat = att_ref[pl.ds((j - 7) * half, half), :]
        proj = jnp.dot(cat, wpt_sc_ref[...], preferred_element_type=jnp.float32)
        o_ref[...] = proj + bp_ref[...].astype(jnp.float32)


@functools.partial(jax.jit, static_argnames=("num_heads", "seqs_per_block"))
def _mha(x, wqkv_bf, wpt_bf, bp_f32, *, num_heads, seqs_per_block):
    B, T, C = x.shape
    BT = B * T
    n_strips = 6
    scols = 3 * C // n_strips
    n_steps = 9

    body = functools.partial(_mha_body, num_heads=num_heads, seq_len=T,
                             seqs_per_block=seqs_per_block, n_strips=n_strips)

    def _strip_col(j):
        jj = jnp.minimum(j, n_strips - 1)
        return jnp.where(jj < 3, 2 * jj, 2 * jj - 5)

    out = pl.pallas_call(
        body,
        out_shape=jax.ShapeDtypeStruct((BT, C), jnp.float32),
        grid=(n_steps,),
        in_specs=[
            pl.BlockSpec((BT, C), lambda j: (0, 0)),
            pl.BlockSpec((C, scols), lambda j: (0, _strip_col(j))),
            pl.BlockSpec((C // n_strips, C),
                         lambda j: (jnp.minimum(j, n_strips - 1), 0)),
            pl.BlockSpec((1, C), lambda j: (0, 0)),
        ],
        out_specs=pl.BlockSpec(
            (BT // 2, C), lambda j: (jnp.maximum(j - 7, 0), 0)),
        scratch_shapes=[
            pltpu.VMEM((BT, C), jnp.bfloat16),
            pltpu.VMEM((n_strips, BT, scols), jnp.bfloat16),
            pltpu.VMEM((C, C), jnp.bfloat16),
            pltpu.VMEM((BT, C), jnp.bfloat16),
        ],
        compiler_params=pltpu.CompilerParams(
            dimension_semantics=("arbitrary",)),
        name="mha_stream",
    )(x.reshape(BT, C), wqkv_bf, wpt_bf, bp_f32)

    return out.reshape(B, T, C)


def kernel(x, wqkv_bf, wpt_bf, bp_f32):
    return _mha(x, wqkv_bf, wpt_bf, bp_f32, num_heads=12, seqs_per_block=2)


# speedup vs baseline: 1.0372x; 1.0372x over previous
import functools

import jax
import jax.numpy as jnp
from jax.experimental import pallas as pl
from jax.experimental.pallas import tpu as pltpu


def _mha_body(x_ref, wqkv_ref, wpt_ref, bp_ref, o_ref, *, num_heads, seq_len):
    R, C = x_ref.shape
    hs = C // num_heads
    T = seq_len

    x = x_ref[...].astype(jnp.bfloat16)

    qkv = jnp.dot(x, wqkv_ref[...], preferred_element_type=jnp.float32)
    qkv = qkv.astype(jnp.bfloat16)

    row = jax.lax.broadcasted_iota(jnp.int32, (R, R), 0)
    col = jax.lax.broadcasted_iota(jnp.int32, (R, R), 1)
    causal = (col <= row) & ((row // T) == (col // T))
    neg_big = jnp.float32(-1e30)

    head_outs = []
    for h in range(num_heads):
        q = qkv[:, h * hs:(h + 1) * hs]
        k = qkv[:, C + h * hs:C + (h + 1) * hs]
        v = qkv[:, 2 * C + h * hs:2 * C + (h + 1) * hs]

        s = jax.lax.dot_general(q, k,
                                dimension_numbers=(((1,), (1,)), ((), ())),
                                preferred_element_type=jnp.float32)
        s = jnp.where(causal, s, neg_big)
        s = s - jnp.max(s, axis=-1, keepdims=True)
        p = jnp.exp(s)
        p = p * pl.reciprocal(jnp.sum(p, axis=-1, keepdims=True), approx=True)

        head_outs.append(jnp.dot(p.astype(jnp.bfloat16), v,
                                 preferred_element_type=jnp.float32))

    cat = jnp.concatenate(head_outs, axis=-1).astype(jnp.bfloat16)
    proj = jnp.dot(cat, wpt_ref[...], preferred_element_type=jnp.float32)
    o_ref[...] = (proj + bp_ref[...].astype(jnp.float32)).astype(o_ref.dtype)


@functools.partial(jax.jit, static_argnames=("num_heads", "seqs_per_step"))
def _mha(x, wqkv_bf, wpt_bf, bp_f32, *, num_heads, seqs_per_step):
    B, T, C = x.shape
    R = seqs_per_step * T
    n_steps = B // seqs_per_step

    body = functools.partial(_mha_body, num_heads=num_heads, seq_len=T)
    out = pl.pallas_call(
        body,
        out_shape=jax.ShapeDtypeStruct((B * T, C), jnp.float32),
        grid=(n_steps,),
        in_specs=[
            pl.BlockSpec((R, C), lambda i: (i, 0)),
            pl.BlockSpec((C, 3 * C), lambda i: (0, 0)),
            pl.BlockSpec((C, C), lambda i: (0, 0)),
            pl.BlockSpec((1, C), lambda i: (0, 0)),
        ],
        out_specs=pl.BlockSpec((R, C), lambda i: (i, 0)),
        compiler_params=pltpu.CompilerParams(
            dimension_semantics=("parallel",)),
        name="mha_blockdiag",
    )(x.reshape(B * T, C), wqkv_bf, wpt_bf, bp_f32)

    return out.reshape(B, T, C)


def kernel(x, wqkv_bf, wpt_bf, bp_f32):
    return _mha(x, wqkv_bf, wpt_bf, bp_f32, num_heads=12, seqs_per_step=2)
